# baseline (speedup 1.0000x reference)
import functools
import math

import jax
import jax.numpy as jnp
from jax import lax
from jax.experimental import pallas as pl
from jax.experimental.pallas import tpu as pltpu
from jax.experimental.pallas import tpu_sc as plsc

N = 10000
E = 320000
NFEAT = 128
NHID = 128
NCLASS = 64
NLAYERS = 8
LAMDA = 0.5
ALPHA = 0.1

NC = 2
NS = 16
FH = NHID // NC
CHUNK = 128
NBUF = 8
CPT = -(-E // (NS * CHUNK * NBUF)) * NBUF
EPT = CPT * CHUNK
E_PAD = EPT * NS
NCH = E_PAD // CHUNK
NGRP = CPT // NBUF
NACC = N + 8
RPT = 632
RPT_LAST_OUT = N - (NS - 1) * RPT
RPT_LAST_ZERO = NACC - (NS - 1) * RPT
LANES = 16


def _spmm_sc_body(gt_hbm, col2_hbm, row_hbm, out_hbm, cidx_v, ridx_v, rows_v, zbuf_v, acc_sh, gsems):
    c = lax.axis_index("c")
    s = lax.axis_index("s")

    zeros = jnp.zeros((LANES,), jnp.float32)

    def zero_row(i, _):
        for j in range(FH // LANES):
            zbuf_v[i, pl.ds(j * LANES, LANES)] = zeros
        return 0

    lax.fori_loop(0, CHUNK, zero_row, 0)

    def zero_slab(base, nrows):
        nfull = nrows // CHUNK
        for k in range(nfull):
            pltpu.sync_copy(zbuf_v, acc_sh.at[pl.ds(base + k * CHUNK, CHUNK)])
        rem = nrows - nfull * CHUNK
        if rem:
            pltpu.sync_copy(
                zbuf_v.at[pl.ds(0, rem)], acc_sh.at[pl.ds(base + nfull * CHUNK, rem)]
            )

    @pl.when(s < NS - 1)
    def _():
        zero_slab(s * RPT, RPT)

    @pl.when(s == NS - 1)
    def _():
        zero_slab((NS - 1) * RPT, RPT_LAST_ZERO)

    plsc.subcore_barrier()

    cbase = s * CPT

    def body(g, _):
        base = cbase + g * NBUF
        pltpu.sync_copy(col2_hbm.at[c, pl.ds(base, NBUF)], cidx_v)
        pltpu.sync_copy(row_hbm.at[pl.ds(base, NBUF)], ridx_v)
        gd = [
            pltpu.async_copy(gt_hbm.at[cidx_v.at[b]], rows_v.at[b], gsems[b])
            for b in range(NBUF)
        ]
        for b in range(NBUF):
            gd[b].wait()
            pltpu.sync_copy(rows_v.at[b], acc_sh.at[ridx_v.at[b]], add=True)
        return 0

    lax.fori_loop(0, NGRP, body, 0)

    plsc.subcore_barrier()

    @pl.when(s < NS - 1)
    def _():
        ob = s * RPT
        pltpu.sync_copy(acc_sh.at[pl.ds(ob, RPT)], out_hbm.at[c, pl.ds(ob, RPT)])

    @pl.when(s == NS - 1)
    def _():
        ob = (NS - 1) * RPT
        pltpu.sync_copy(
            acc_sh.at[pl.ds(ob, RPT_LAST_OUT)], out_hbm.at[c, pl.ds(ob, RPT_LAST_OUT)]
        )


@functools.cache
def _get_spmm_sc():
    return pl.kernel(
        _spmm_sc_body,
        out_type=jax.ShapeDtypeStruct((NC, N, FH), jnp.float32),
        mesh=plsc.VectorSubcoreMesh(
            core_axis_name="c", subcore_axis_name="s", num_cores=NC, num_subcores=NS
        ),
        scratch_types=[
            pltpu.VMEM((NBUF, CHUNK), jnp.int32),
            pltpu.VMEM((NBUF, CHUNK), jnp.int32),
            pltpu.VMEM((NBUF, CHUNK, FH), jnp.float32),
            pltpu.VMEM((CHUNK, FH), jnp.float32),
            pltpu.VMEM_SHARED((NACC, FH), jnp.float32),
            [pltpu.SemaphoreType.DMA] * NBUF,
        ],
        compiler_params=pltpu.CompilerParams(use_tc_tiling_on_sc=False),
    )


def _spmm_sc(gp, col2, rowp):
    return _get_spmm_sc()(gp.reshape(2 * N, FH), col2, rowp)


_BT = 1000


def _pre_body(x_ref, w_ref, b_ref, dis_ref, h0_ref, gp_ref):
    h = jnp.dot(x_ref[...], w_ref[...], preferred_element_type=jnp.float32)
    h = jnp.maximum(h + b_ref[...], 0.0)
    h0_ref[...] = h
    g = h * dis_ref[...]
    gp_ref[...] = g.reshape(_BT, NC, FH)


def _dense_pre(x, w_t, b, dis):
    return pl.pallas_call(
        _pre_body,
        grid=(N // _BT,),
        in_specs=[
            pl.BlockSpec((_BT, NFEAT), lambda i: (i, 0)),
            pl.BlockSpec((NFEAT, NHID), lambda i: (0, 0)),
            pl.BlockSpec((1, NHID), lambda i: (0, 0)),
            pl.BlockSpec((_BT, 1), lambda i: (i, 0)),
        ],
        out_specs=[
            pl.BlockSpec((_BT, NHID), lambda i: (i, 0)),
            pl.BlockSpec((_BT, NC, FH), lambda i: (i, 0, 0)),
        ],
        out_shape=[
            jax.ShapeDtypeStruct((N, NHID), jnp.float32),
            jax.ShapeDtypeStruct((N, NC, FH), jnp.float32),
        ],
    )(x, w_t, b, dis)


def _layer_body(theta, p0_ref, p1_ref, gp_ref, h0_ref, dis_ref, w_ref, h_ref, gn_ref):
    dis = dis_ref[...]
    p0 = p0_ref[0]
    p1 = p1_ref[0]
    g0 = gp_ref[:, 0, :]
    g1 = gp_ref[:, 1, :]
    s0 = (1.0 - ALPHA) * (dis * (p0 + g0)) + ALPHA * h0_ref[:, :FH]
    s1 = (1.0 - ALPHA) * (dis * (p1 + g1)) + ALPHA * h0_ref[:, FH:]
    mm = jnp.dot(s0, w_ref[:FH, :], preferred_element_type=jnp.float32)
    mm = mm + jnp.dot(s1, w_ref[FH:, :], preferred_element_type=jnp.float32)
    h0h = jnp.maximum(theta * mm[:, :FH] + (1.0 - theta) * s0, 0.0)
    h1h = jnp.maximum(theta * mm[:, FH:] + (1.0 - theta) * s1, 0.0)
    h_ref[...] = jnp.concatenate([h0h, h1h], axis=1)
    gn_ref[...] = jnp.stack([h0h * dis, h1h * dis], axis=1)


def _dense_layer(theta, p, gp, h0, dis, w):
    return pl.pallas_call(
        functools.partial(_layer_body, theta),
        grid=(N // _BT,),
        in_specs=[
            pl.BlockSpec((1, _BT, FH), lambda i: (0, i, 0)),
            pl.BlockSpec((1, _BT, FH), lambda i: (1, i, 0)),
            pl.BlockSpec((_BT, NC, FH), lambda i: (i, 0, 0)),
            pl.BlockSpec((_BT, NHID), lambda i: (i, 0)),
            pl.BlockSpec((_BT, 1), lambda i: (i, 0)),
            pl.BlockSpec((NHID, NHID), lambda i: (0, 0)),
        ],
        out_specs=[
            pl.BlockSpec((_BT, NHID), lambda i: (i, 0)),
            pl.BlockSpec((_BT, NC, FH), lambda i: (i, 0, 0)),
        ],
        out_shape=[
            jax.ShapeDtypeStruct((N, NHID), jnp.float32),
            jax.ShapeDtypeStruct((N, NC, FH), jnp.float32),
        ],
    )(p, p, gp, h0, dis, w)


def _final_body(h_ref, w_ref, b_ref, o_ref):
    o = jnp.dot(h_ref[...], w_ref[...], preferred_element_type=jnp.float32)
    o_ref[...] = o + b_ref[...]


def _dense_final(h, w_t, b):
    return pl.pallas_call(
        _final_body,
        grid=(N // _BT,),
        in_specs=[
            pl.BlockSpec((_BT, NHID), lambda i: (i, 0)),
            pl.BlockSpec((NHID, NCLASS), lambda i: (0, 0)),
            pl.BlockSpec((1, NCLASS), lambda i: (0, 0)),
        ],
        out_specs=pl.BlockSpec((_BT, NCLASS), lambda i: (i, 0)),
        out_shape=jax.ShapeDtypeStruct((N, NCLASS), jnp.float32),
    )(h, w_t, b)


def kernel(x, edge_idx, fc0_w, fc0_b, convs_w, fc1_w, fc1_b):
    row = edge_idx[0].astype(jnp.int32)
    col = edge_idx[1].astype(jnp.int32)
    self_m = row == col
    w_e = jnp.where(self_m, 0.0, 1.0).astype(jnp.float32)
    deg = jnp.zeros((N,), jnp.float32).at[row].add(w_e) + 1.0
    dis = lax.rsqrt(deg).reshape(N, 1)

    row_r = jnp.where(self_m, N, row)
    rowf = jnp.concatenate([row_r, jnp.full((E_PAD - E,), N, jnp.int32)])
    colf = jnp.concatenate([col, jnp.zeros((E_PAD - E,), jnp.int32)])
    rowp = rowf.reshape(NCH, CHUNK)
    col2 = jnp.stack([2 * colf, 2 * colf + 1]).reshape(NC, NCH, CHUNK)

    h0, gp = _dense_pre(x, fc0_w.T, fc0_b.reshape(1, NHID), dis)
    h = h0
    for i in range(NLAYERS):
        theta = math.log(LAMDA / (i + 1) + 1.0)
        p = _spmm_sc(gp, col2, rowp)
        h, gp = _dense_layer(theta, p, gp, h0, dis, convs_w[i])
    return _dense_final(h, fc1_w.T, fc1_b.reshape(1, NCLASS))


# speedup vs baseline: 1.9910x; 1.9910x over previous
import functools
import math

import jax
import jax.numpy as jnp
from jax import lax
from jax.experimental import pallas as pl
from jax.experimental.pallas import tpu as pltpu
from jax.experimental.pallas import tpu_sc as plsc

N = 10000
E = 320000
NFEAT = 128
NHID = 128
NCLASS = 64
NLAYERS = 8
LAMDA = 0.5
ALPHA = 0.1

NC = 2
NS = 16
FH = NHID // NC
CHUNK = 128
NBUF = 4
CPT = -(-E // (NS * CHUNK * NBUF)) * NBUF
EPT = CPT * CHUNK
E_PAD = EPT * NS
NCH = E_PAD // CHUNK
NGRP = CPT // NBUF
NACC = N + 8
RPT = 632
RPT_LAST_OUT = N - (NS - 1) * RPT
RPT_LAST_ZERO = NACC - (NS - 1) * RPT
LANES = 16


def _spmm_sc_body(
    gt_hbm, col_hbm, row_hbm, out_hbm, cidx_v, ridx_v, rows_v, zbuf_v, tab_sh, acc_sh, gsems
):
    c = lax.axis_index("c")
    s = lax.axis_index("s")

    zeros = jnp.zeros((LANES,), jnp.float32)

    def zero_row(i, _):
        for j in range(FH // LANES):
            zbuf_v[i, pl.ds(j * LANES, LANES)] = zeros
        return 0

    lax.fori_loop(0, CHUNK, zero_row, 0)

    def prep_slab(base, nrows, nzero):
        nfull = nzero // CHUNK
        for k in range(nfull):
            pltpu.sync_copy(zbuf_v, acc_sh.at[pl.ds(base + k * CHUNK, CHUNK)])
        rem = nzero - nfull * CHUNK
        if rem:
            pltpu.sync_copy(
                zbuf_v.at[pl.ds(0, rem)], acc_sh.at[pl.ds(base + nfull * CHUNK, rem)]
            )
        pltpu.sync_copy(gt_hbm.at[c, pl.ds(base, nrows)], tab_sh.at[pl.ds(base, nrows)])

    @pl.when(s < NS - 1)
    def _():
        prep_slab(s * RPT, RPT, RPT)

    @pl.when(s == NS - 1)
    def _():
        prep_slab((NS - 1) * RPT, RPT_LAST_OUT, RPT_LAST_ZERO)

    plsc.subcore_barrier()

    cbase = s * CPT

    def body(g, _):
        base = cbase + g * NBUF
        pltpu.sync_copy(col_hbm.at[pl.ds(base, NBUF)], cidx_v)
        pltpu.sync_copy(row_hbm.at[pl.ds(base, NBUF)], ridx_v)
        gd = [
            pltpu.async_copy(tab_sh.at[cidx_v.at[b]], rows_v.at[b], gsems[b])
            for b in range(NBUF)
        ]
        for b in range(NBUF):
            gd[b].wait()
            pltpu.sync_copy(rows_v.at[b], acc_sh.at[ridx_v.at[b]], add=True)
        return 0

    lax.fori_loop(0, NGRP, body, 0)

    plsc.subcore_barrier()

    @pl.when(s < NS - 1)
    def _():
        ob = s * RPT
        pltpu.sync_copy(acc_sh.at[pl.ds(ob, RPT)], out_hbm.at[c, pl.ds(ob, RPT)])

    @pl.when(s == NS - 1)
    def _():
        ob = (NS - 1) * RPT
        pltpu.sync_copy(
            acc_sh.at[pl.ds(ob, RPT_LAST_OUT)], out_hbm.at[c, pl.ds(ob, RPT_LAST_OUT)]
        )


@functools.cache
def _get_spmm_sc():
    return pl.kernel(
        _spmm_sc_body,
        out_type=jax.ShapeDtypeStruct((NC, N, FH), jnp.float32),
        mesh=plsc.VectorSubcoreMesh(
            core_axis_name="c", subcore_axis_name="s", num_cores=NC, num_subcores=NS
        ),
        scratch_types=[
            pltpu.VMEM((NBUF, CHUNK), jnp.int32),
            pltpu.VMEM((NBUF, CHUNK), jnp.int32),
            pltpu.VMEM((NBUF, CHUNK, FH), jnp.float32),
            pltpu.VMEM((CHUNK, FH), jnp.float32),
            pltpu.VMEM_SHARED((N, FH), jnp.float32),
            pltpu.VMEM_SHARED((NACC, FH), jnp.float32),
            [pltpu.SemaphoreType.DMA] * NBUF,
        ],
        compiler_params=pltpu.CompilerParams(use_tc_tiling_on_sc=False),
    )


def _spmm_sc(gt, colp, rowp):
    return _get_spmm_sc()(gt, colp, rowp)


_BT = 1000


def _pre_body(x_ref, w_ref, b_ref, dis_ref, h0_ref, gp_ref):
    h = jnp.dot(x_ref[...], w_ref[...], preferred_element_type=jnp.float32)
    h = jnp.maximum(h + b_ref[...], 0.0)
    h0_ref[...] = h
    g = h * dis_ref[...]
    gp_ref[0] = g[:, :FH]
    gp_ref[1] = g[:, FH:]


def _dense_pre(x, w_t, b, dis):
    return pl.pallas_call(
        _pre_body,
        grid=(N // _BT,),
        in_specs=[
            pl.BlockSpec((_BT, NFEAT), lambda i: (i, 0)),
            pl.BlockSpec((NFEAT, NHID), lambda i: (0, 0)),
            pl.BlockSpec((1, NHID), lambda i: (0, 0)),
            pl.BlockSpec((_BT, 1), lambda i: (i, 0)),
        ],
        out_specs=[
            pl.BlockSpec((_BT, NHID), lambda i: (i, 0)),
            pl.BlockSpec((NC, _BT, FH), lambda i: (0, i, 0)),
        ],
        out_shape=[
            jax.ShapeDtypeStruct((N, NHID), jnp.float32),
            jax.ShapeDtypeStruct((NC, N, FH), jnp.float32),
        ],
    )(x, w_t, b, dis)


def _layer_body(theta, p_ref, gp_ref, h0_ref, dis_ref, w_ref, h_ref, gn_ref):
    dis = dis_ref[...]
    s0 = (1.0 - ALPHA) * (dis * (p_ref[0] + gp_ref[0])) + ALPHA * h0_ref[:, :FH]
    s1 = (1.0 - ALPHA) * (dis * (p_ref[1] + gp_ref[1])) + ALPHA * h0_ref[:, FH:]
    mm = jnp.dot(s0, w_ref[:FH, :], preferred_element_type=jnp.float32)
    mm = mm + jnp.dot(s1, w_ref[FH:, :], preferred_element_type=jnp.float32)
    h0h = jnp.maximum(theta * mm[:, :FH] + (1.0 - theta) * s0, 0.0)
    h1h = jnp.maximum(theta * mm[:, FH:] + (1.0 - theta) * s1, 0.0)
    h_ref[...] = jnp.concatenate([h0h, h1h], axis=1)
    gn_ref[0] = h0h * dis
    gn_ref[1] = h1h * dis


def _dense_layer(theta, p, gp, h0, dis, w):
    return pl.pallas_call(
        functools.partial(_layer_body, theta),
        grid=(N // _BT,),
        in_specs=[
            pl.BlockSpec((NC, _BT, FH), lambda i: (0, i, 0)),
            pl.BlockSpec((NC, _BT, FH), lambda i: (0, i, 0)),
            pl.BlockSpec((_BT, NHID), lambda i: (i, 0)),
            pl.BlockSpec((_BT, 1), lambda i: (i, 0)),
            pl.BlockSpec((NHID, NHID), lambda i: (0, 0)),
        ],
        out_specs=[
            pl.BlockSpec((_BT, NHID), lambda i: (i, 0)),
            pl.BlockSpec((NC, _BT, FH), lambda i: (0, i, 0)),
        ],
        out_shape=[
            jax.ShapeDtypeStruct((N, NHID), jnp.float32),
            jax.ShapeDtypeStruct((NC, N, FH), jnp.float32),
        ],
    )(p, gp, h0, dis, w)


def _final_body(h_ref, w_ref, b_ref, o_ref):
    o = jnp.dot(h_ref[...], w_ref[...], preferred_element_type=jnp.float32)
    o_ref[...] = o + b_ref[...]


def _dense_final(h, w_t, b):
    return pl.pallas_call(
        _final_body,
        grid=(N // _BT,),
        in_specs=[
            pl.BlockSpec((_BT, NHID), lambda i: (i, 0)),
            pl.BlockSpec((NHID, NCLASS), lambda i: (0, 0)),
            pl.BlockSpec((1, NCLASS), lambda i: (0, 0)),
        ],
        out_specs=pl.BlockSpec((_BT, NCLASS), lambda i: (i, 0)),
        out_shape=jax.ShapeDtypeStruct((N, NCLASS), jnp.float32),
    )(h, w_t, b)


def kernel(x, edge_idx, fc0_w, fc0_b, convs_w, fc1_w, fc1_b):
    row = edge_idx[0].astype(jnp.int32)
    col = edge_idx[1].astype(jnp.int32)
    self_m = row == col
    w_e = jnp.where(self_m, 0.0, 1.0).astype(jnp.float32)
    deg = jnp.zeros((N,), jnp.float32).at[row].add(w_e) + 1.0
    dis = lax.rsqrt(deg).reshape(N, 1)

    row_r = jnp.where(self_m, N, row)
    rowf = jnp.concatenate([row_r, jnp.full((E_PAD - E,), N, jnp.int32)])
    colf = jnp.concatenate([col, jnp.zeros((E_PAD - E,), jnp.int32)])
    rowp = rowf.reshape(NCH, CHUNK)
    colp = colf.reshape(NCH, CHUNK)

    h0, gp = _dense_pre(x, fc0_w.T, fc0_b.reshape(1, NHID), dis)
    h = h0
    for i in range(NLAYERS):
        theta = math.log(LAMDA / (i + 1) + 1.0)
        p = _spmm_sc(gp, colp, rowp)
        h, gp = _dense_layer(theta, p, gp, h0, dis, convs_w[i])
    return _dense_final(h, fc1_w.T, fc1_b.reshape(1, NCLASS))


# speedup vs baseline: 2.4137x; 1.2123x over previous
import functools
import math

import jax
import jax.numpy as jnp
from jax import lax
from jax.experimental import pallas as pl
from jax.experimental.pallas import tpu as pltpu
from jax.experimental.pallas import tpu_sc as plsc

N = 10000
E = 320000
NFEAT = 128
NHID = 128
NCLASS = 64
NLAYERS = 8
LAMDA = 0.5
ALPHA = 0.1

NC = 2
NS = 16
FH = NHID // NC
CHUNK = 128
NBUF = 4
CPT = -(-E // (NS * CHUNK * NBUF)) * NBUF
EPT = CPT * CHUNK
E_PAD = EPT * NS
NCH = E_PAD // CHUNK
NGRP = CPT // NBUF
NACC = N + 8
RPT = 632
RPT_LAST_OUT = N - (NS - 1) * RPT
RPT_LAST_ZERO = NACC - (NS - 1) * RPT
LANES = 16


def _spmm_sc_body(
    gt_hbm, col_hbm, row_hbm, out_hbm, cidx_v, ridx_v, rows_v, zbuf_v, tab_sh, acc_sh, gsems
):
    c = lax.axis_index("c")
    s = lax.axis_index("s")

    zeros = jnp.zeros((LANES,), jnp.float32)

    def zero_row(i, _):
        for j in range(FH // LANES):
            zbuf_v[i, pl.ds(j * LANES, LANES)] = zeros
        return 0

    lax.fori_loop(0, CHUNK, zero_row, 0)

    def prep_slab(base, nrows, nzero):
        nfull = nzero // CHUNK
        for k in range(nfull):
            pltpu.sync_copy(zbuf_v, acc_sh.at[pl.ds(base + k * CHUNK, CHUNK)])
        rem = nzero - nfull * CHUNK
        if rem:
            pltpu.sync_copy(
                zbuf_v.at[pl.ds(0, rem)], acc_sh.at[pl.ds(base + nfull * CHUNK, rem)]
            )
        pltpu.sync_copy(gt_hbm.at[c, pl.ds(base, nrows)], tab_sh.at[pl.ds(base, nrows)])

    @pl.when(s < NS - 1)
    def _():
        prep_slab(s * RPT, RPT, RPT)

    @pl.when(s == NS - 1)
    def _():
        prep_slab((NS - 1) * RPT, RPT_LAST_OUT, RPT_LAST_ZERO)

    plsc.subcore_barrier()

    cbase = s * CPT

    def body(g, _):
        base = cbase + g * NBUF
        pltpu.sync_copy(col_hbm.at[pl.ds(base, NBUF)], cidx_v)
        pltpu.sync_copy(row_hbm.at[pl.ds(base, NBUF)], ridx_v)
        gd = [
            pltpu.async_copy(tab_sh.at[cidx_v.at[b]], rows_v.at[b], gsems[b])
            for b in range(NBUF)
        ]
        for b in range(NBUF):
            gd[b].wait()
            pltpu.sync_copy(rows_v.at[b], acc_sh.at[ridx_v.at[b]], add=True)
        return 0

    lax.fori_loop(0, NGRP, body, 0)

    plsc.subcore_barrier()

    @pl.when(s < NS - 1)
    def _():
        ob = s * RPT
        pltpu.sync_copy(acc_sh.at[pl.ds(ob, RPT)], out_hbm.at[c, pl.ds(ob, RPT)])

    @pl.when(s == NS - 1)
    def _():
        ob = (NS - 1) * RPT
        pltpu.sync_copy(
            acc_sh.at[pl.ds(ob, RPT_LAST_OUT)], out_hbm.at[c, pl.ds(ob, RPT_LAST_OUT)]
        )


@functools.cache
def _get_spmm_sc():
    return pl.kernel(
        _spmm_sc_body,
        out_type=jax.ShapeDtypeStruct((NC, N, FH), jnp.float32),
        mesh=plsc.VectorSubcoreMesh(
            core_axis_name="c", subcore_axis_name="s", num_cores=NC, num_subcores=NS
        ),
        scratch_types=[
            pltpu.VMEM((NBUF, CHUNK), jnp.int32),
            pltpu.VMEM((NBUF, CHUNK), jnp.int32),
            pltpu.VMEM((NBUF, CHUNK, FH), jnp.float32),
            pltpu.VMEM((CHUNK, FH), jnp.float32),
            pltpu.VMEM_SHARED((N, FH), jnp.float32),
            pltpu.VMEM_SHARED((NACC, FH), jnp.float32),
            [pltpu.SemaphoreType.DMA] * NBUF,
        ],
        compiler_params=pltpu.CompilerParams(use_tc_tiling_on_sc=False),
    )


def _spmm_sc(gt, colp, rowp):
    return _get_spmm_sc()(gt, colp, rowp)


DW = LANES
CPT_D = NCH // NC // NS


def _deg_sc_body(row_hbm, out_hbm, ridx_v, ones_v, zbuf_v, acc_sh):
    c = lax.axis_index("c")
    s = lax.axis_index("s")

    ones = jnp.full((LANES,), 1.0, jnp.float32)
    zeros = jnp.zeros((LANES,), jnp.float32)

    def fill_row(i, _):
        ones_v[i, pl.ds(0, LANES)] = ones
        zbuf_v[i, pl.ds(0, LANES)] = zeros
        return 0

    lax.fori_loop(0, CHUNK, fill_row, 0)

    def zero_slab(base, nzero):
        nfull = nzero // CHUNK
        for k in range(nfull):
            pltpu.sync_copy(zbuf_v, acc_sh.at[pl.ds(base + k * CHUNK, CHUNK)])
        rem = nzero - nfull * CHUNK
        if rem:
            pltpu.sync_copy(
                zbuf_v.at[pl.ds(0, rem)], acc_sh.at[pl.ds(base + nfull * CHUNK, rem)]
            )

    @pl.when(s < NS - 1)
    def _():
        zero_slab(s * RPT, RPT)

    @pl.when(s == NS - 1)
    def _():
        zero_slab((NS - 1) * RPT, RPT_LAST_ZERO)

    plsc.subcore_barrier()

    cbase = (c * NS + s) * CPT_D
    pltpu.sync_copy(row_hbm.at[pl.ds(cbase, CPT_D)], ridx_v)

    def body(k, _):
        pltpu.sync_copy(ones_v, acc_sh.at[ridx_v.at[k]], add=True)
        return 0

    lax.fori_loop(0, CPT_D, body, 0)

    plsc.subcore_barrier()

    @pl.when(s < NS - 1)
    def _():
        ob = s * RPT
        pltpu.sync_copy(acc_sh.at[pl.ds(ob, RPT)], out_hbm.at[c, pl.ds(ob, RPT)])

    @pl.when(s == NS - 1)
    def _():
        ob = (NS - 1) * RPT
        pltpu.sync_copy(
            acc_sh.at[pl.ds(ob, RPT_LAST_OUT)], out_hbm.at[c, pl.ds(ob, RPT_LAST_OUT)]
        )


@functools.cache
def _get_deg_sc():
    return pl.kernel(
        _deg_sc_body,
        out_type=jax.ShapeDtypeStruct((NC, N, DW), jnp.float32),
        mesh=plsc.VectorSubcoreMesh(
            core_axis_name="c", subcore_axis_name="s", num_cores=NC, num_subcores=NS
        ),
        scratch_types=[
            pltpu.VMEM((CPT_D, CHUNK), jnp.int32),
            pltpu.VMEM((CHUNK, DW), jnp.float32),
            pltpu.VMEM((CHUNK, DW), jnp.float32),
            pltpu.VMEM_SHARED((NACC, DW), jnp.float32),
        ],
        compiler_params=pltpu.CompilerParams(use_tc_tiling_on_sc=False),
    )


_BT = 1000


def _pre_body(x_ref, w_ref, b_ref, dp_ref, h0_ref, gp_ref, dis_ref):
    deg = dp_ref[0, :, 0:1] + dp_ref[1, :, 0:1] + 1.0
    dis = lax.rsqrt(deg)
    dis_ref[...] = dis
    h = jnp.dot(x_ref[...], w_ref[...], preferred_element_type=jnp.float32)
    h = jnp.maximum(h + b_ref[...], 0.0)
    h0_ref[...] = h
    g = h * dis
    gp_ref[0] = g[:, :FH]
    gp_ref[1] = g[:, FH:]


def _dense_pre(x, w_t, b, dp):
    return pl.pallas_call(
        _pre_body,
        grid=(N // _BT,),
        in_specs=[
            pl.BlockSpec((_BT, NFEAT), lambda i: (i, 0)),
            pl.BlockSpec((NFEAT, NHID), lambda i: (0, 0)),
            pl.BlockSpec((1, NHID), lambda i: (0, 0)),
            pl.BlockSpec((NC, _BT, DW), lambda i: (0, i, 0)),
        ],
        out_specs=[
            pl.BlockSpec((_BT, NHID), lambda i: (i, 0)),
            pl.BlockSpec((NC, _BT, FH), lambda i: (0, i, 0)),
            pl.BlockSpec((_BT, 1), lambda i: (i, 0)),
        ],
        out_shape=[
            jax.ShapeDtypeStruct((N, NHID), jnp.float32),
            jax.ShapeDtypeStruct((NC, N, FH), jnp.float32),
            jax.ShapeDtypeStruct((N, 1), jnp.float32),
        ],
    )(x, w_t, b, dp)


def _layer_body(theta, emit_h, p_ref, gp_ref, h0_ref, dis_ref, w_ref, *out_refs):
    dis = dis_ref[...]
    s0 = (1.0 - ALPHA) * (dis * (p_ref[0] + gp_ref[0])) + ALPHA * h0_ref[:, :FH]
    s1 = (1.0 - ALPHA) * (dis * (p_ref[1] + gp_ref[1])) + ALPHA * h0_ref[:, FH:]
    mm = jnp.dot(s0, w_ref[:FH, :], preferred_element_type=jnp.float32)
    mm = mm + jnp.dot(s1, w_ref[FH:, :], preferred_element_type=jnp.float32)
    h0h = jnp.maximum(theta * mm[:, :FH] + (1.0 - theta) * s0, 0.0)
    h1h = jnp.maximum(theta * mm[:, FH:] + (1.0 - theta) * s1, 0.0)
    gn_ref = out_refs[0]
    gn_ref[0] = h0h * dis
    gn_ref[1] = h1h * dis
    if emit_h:
        out_refs[1][...] = jnp.concatenate([h0h, h1h], axis=1)


def _dense_layer(theta, emit_h, p, gp, h0, dis, w):
    out_specs = [pl.BlockSpec((NC, _BT, FH), lambda i: (0, i, 0))]
    out_shape = [jax.ShapeDtypeStruct((NC, N, FH), jnp.float32)]
    if emit_h:
        out_specs.append(pl.BlockSpec((_BT, NHID), lambda i: (i, 0)))
        out_shape.append(jax.ShapeDtypeStruct((N, NHID), jnp.float32))
    return pl.pallas_call(
        functools.partial(_layer_body, theta, emit_h),
        grid=(N // _BT,),
        in_specs=[
            pl.BlockSpec((NC, _BT, FH), lambda i: (0, i, 0)),
            pl.BlockSpec((NC, _BT, FH), lambda i: (0, i, 0)),
            pl.BlockSpec((_BT, NHID), lambda i: (i, 0)),
            pl.BlockSpec((_BT, 1), lambda i: (i, 0)),
            pl.BlockSpec((NHID, NHID), lambda i: (0, 0)),
        ],
        out_specs=out_specs,
        out_shape=out_shape,
    )(p, gp, h0, dis, w)


def _final_body(h_ref, w_ref, b_ref, o_ref):
    o = jnp.dot(h_ref[...], w_ref[...], preferred_element_type=jnp.float32)
    o_ref[...] = o + b_ref[...]


def _dense_final(h, w_t, b):
    return pl.pallas_call(
        _final_body,
        grid=(N // _BT,),
        in_specs=[
            pl.BlockSpec((_BT, NHID), lambda i: (i, 0)),
            pl.BlockSpec((NHID, NCLASS), lambda i: (0, 0)),
            pl.BlockSpec((1, NCLASS), lambda i: (0, 0)),
        ],
        out_specs=pl.BlockSpec((_BT, NCLASS), lambda i: (i, 0)),
        out_shape=jax.ShapeDtypeStruct((N, NCLASS), jnp.float32),
    )(h, w_t, b)


def kernel(x, edge_idx, fc0_w, fc0_b, convs_w, fc1_w, fc1_b):
    row = edge_idx[0].astype(jnp.int32)
    col = edge_idx[1].astype(jnp.int32)
    self_m = row == col

    row_r = jnp.where(self_m, N, row)
    rowf = jnp.concatenate([row_r, jnp.full((E_PAD - E,), N, jnp.int32)])
    colf = jnp.concatenate([col, jnp.zeros((E_PAD - E,), jnp.int32)])
    rowp = rowf.reshape(NCH, CHUNK)
    colp = colf.reshape(NCH, CHUNK)

    dp = _get_deg_sc()(rowp)
    h0, gp, dis = _dense_pre(x, fc0_w.T, fc0_b.reshape(1, NHID), dp)
    h = h0
    for i in range(NLAYERS):
        theta = math.log(LAMDA / (i + 1) + 1.0)
        p = _spmm_sc(gp, colp, rowp)
        outs = _dense_layer(theta, i == NLAYERS - 1, p, gp, h0, dis, convs_w[i])
        gp = outs[0]
        if i == NLAYERS - 1:
            h = outs[1]
    return _dense_final(h, fc1_w.T, fc1_b.reshape(1, NCLASS))


# speedup vs baseline: 2.6886x; 1.1139x over previous
import functools
import math

import jax
import jax.numpy as jnp
from jax import lax
from jax.experimental import pallas as pl
from jax.experimental.pallas import tpu as pltpu
from jax.experimental.pallas import tpu_sc as plsc

N = 10000
E = 320000
NFEAT = 128
NHID = 128
NCLASS = 64
NLAYERS = 8
LAMDA = 0.5
ALPHA = 0.1

NC = 2
NS = 16
FH = NHID // NC
CHUNK = 128
NBUF = 4
CPT = -(-E // (NS * CHUNK * NBUF)) * NBUF
EPT = CPT * CHUNK
E_PAD = EPT * NS
NCH = E_PAD // CHUNK
NGRP = CPT // NBUF
NACC = N + 8
RPT = 632
RPT_LAST_OUT = N - (NS - 1) * RPT
RPT_LAST_ZERO = NACC - (NS - 1) * RPT
LANES = 16


def _spmm_sc_body(
    gt_hbm, col_hbm, row_hbm, out_hbm, cidx_v, ridx_v, rows_v, zbuf_v, tab_sh, acc_sh, gsems
):
    c = lax.axis_index("c")
    s = lax.axis_index("s")

    zeros = jnp.zeros((LANES,), jnp.float32)

    def zero_row(i, _):
        for j in range(FH // LANES):
            zbuf_v[i, pl.ds(j * LANES, LANES)] = zeros
        return 0

    lax.fori_loop(0, CHUNK, zero_row, 0)

    def prep_slab(base, nrows, nzero):
        nfull = nzero // CHUNK
        for k in range(nfull):
            pltpu.sync_copy(zbuf_v, acc_sh.at[pl.ds(base + k * CHUNK, CHUNK)])
        rem = nzero - nfull * CHUNK
        if rem:
            pltpu.sync_copy(
                zbuf_v.at[pl.ds(0, rem)], acc_sh.at[pl.ds(base + nfull * CHUNK, rem)]
            )
        pltpu.sync_copy(
            gt_hbm.at[pl.ds(base, nrows), pl.ds(c * FH, FH)], tab_sh.at[pl.ds(base, nrows)]
        )

    @pl.when(s < NS - 1)
    def _():
        prep_slab(s * RPT, RPT, RPT)

    @pl.when(s == NS - 1)
    def _():
        prep_slab((NS - 1) * RPT, RPT_LAST_OUT, RPT_LAST_ZERO)

    plsc.subcore_barrier()

    cbase = s * CPT

    def body(g, _):
        base = cbase + g * NBUF
        pltpu.sync_copy(col_hbm.at[pl.ds(base, NBUF)], cidx_v)
        pltpu.sync_copy(row_hbm.at[pl.ds(base, NBUF)], ridx_v)
        gd = [
            pltpu.async_copy(tab_sh.at[cidx_v.at[b]], rows_v.at[b], gsems[b])
            for b in range(NBUF)
        ]
        for b in range(NBUF):
            gd[b].wait()
            pltpu.sync_copy(rows_v.at[b], acc_sh.at[ridx_v.at[b]], add=True)
        return 0

    lax.fori_loop(0, NGRP, body, 0)

    plsc.subcore_barrier()

    @pl.when(s < NS - 1)
    def _():
        ob = s * RPT
        pltpu.sync_copy(
            acc_sh.at[pl.ds(ob, RPT)], out_hbm.at[pl.ds(ob, RPT), pl.ds(c * FH, FH)]
        )

    @pl.when(s == NS - 1)
    def _():
        ob = (NS - 1) * RPT
        pltpu.sync_copy(
            acc_sh.at[pl.ds(ob, RPT_LAST_OUT)],
            out_hbm.at[pl.ds(ob, RPT_LAST_OUT), pl.ds(c * FH, FH)],
        )


@functools.cache
def _get_spmm_sc():
    return pl.kernel(
        _spmm_sc_body,
        out_type=jax.ShapeDtypeStruct((N, NHID), jnp.float32),
        mesh=plsc.VectorSubcoreMesh(
            core_axis_name="c", subcore_axis_name="s", num_cores=NC, num_subcores=NS
        ),
        scratch_types=[
            pltpu.VMEM((NBUF, CHUNK), jnp.int32),
            pltpu.VMEM((NBUF, CHUNK), jnp.int32),
            pltpu.VMEM((NBUF, CHUNK, FH), jnp.float32),
            pltpu.VMEM((CHUNK, FH), jnp.float32),
            pltpu.VMEM_SHARED((N, FH), jnp.float32),
            pltpu.VMEM_SHARED((NACC, FH), jnp.float32),
            [pltpu.SemaphoreType.DMA] * NBUF,
        ],
        compiler_params=pltpu.CompilerParams(use_tc_tiling_on_sc=False),
    )


def _spmm_sc(gt, colp, rowp):
    return _get_spmm_sc()(gt, colp, rowp)


DW = LANES
CPT_D = NCH // NC // NS


def _deg_sc_body(row_hbm, out_hbm, ridx_v, ones_v, zbuf_v, acc_sh):
    c = lax.axis_index("c")
    s = lax.axis_index("s")

    ones = jnp.full((LANES,), 1.0, jnp.float32)
    zeros = jnp.zeros((LANES,), jnp.float32)

    def fill_row(i, _):
        ones_v[i, pl.ds(0, LANES)] = ones
        zbuf_v[i, pl.ds(0, LANES)] = zeros
        return 0

    lax.fori_loop(0, CHUNK, fill_row, 0)

    def zero_slab(base, nzero):
        nfull = nzero // CHUNK
        for k in range(nfull):
            pltpu.sync_copy(zbuf_v, acc_sh.at[pl.ds(base + k * CHUNK, CHUNK)])
        rem = nzero - nfull * CHUNK
        if rem:
            pltpu.sync_copy(
                zbuf_v.at[pl.ds(0, rem)], acc_sh.at[pl.ds(base + nfull * CHUNK, rem)]
            )

    @pl.when(s < NS - 1)
    def _():
        zero_slab(s * RPT, RPT)

    @pl.when(s == NS - 1)
    def _():
        zero_slab((NS - 1) * RPT, RPT_LAST_ZERO)

    plsc.subcore_barrier()

    cbase = (c * NS + s) * CPT_D
    pltpu.sync_copy(row_hbm.at[pl.ds(cbase, CPT_D)], ridx_v)

    def body(k, _):
        pltpu.sync_copy(ones_v, acc_sh.at[ridx_v.at[k]], add=True)
        return 0

    lax.fori_loop(0, CPT_D, body, 0)

    plsc.subcore_barrier()

    @pl.when(s < NS - 1)
    def _():
        ob = s * RPT
        pltpu.sync_copy(acc_sh.at[pl.ds(ob, RPT)], out_hbm.at[c, pl.ds(ob, RPT)])

    @pl.when(s == NS - 1)
    def _():
        ob = (NS - 1) * RPT
        pltpu.sync_copy(
            acc_sh.at[pl.ds(ob, RPT_LAST_OUT)], out_hbm.at[c, pl.ds(ob, RPT_LAST_OUT)]
        )


@functools.cache
def _get_deg_sc():
    return pl.kernel(
        _deg_sc_body,
        out_type=jax.ShapeDtypeStruct((NC, N, DW), jnp.float32),
        mesh=plsc.VectorSubcoreMesh(
            core_axis_name="c", subcore_axis_name="s", num_cores=NC, num_subcores=NS
        ),
        scratch_types=[
            pltpu.VMEM((CPT_D, CHUNK), jnp.int32),
            pltpu.VMEM((CHUNK, DW), jnp.float32),
            pltpu.VMEM((CHUNK, DW), jnp.float32),
            pltpu.VMEM_SHARED((NACC, DW), jnp.float32),
        ],
        compiler_params=pltpu.CompilerParams(use_tc_tiling_on_sc=False),
    )


_BT = 1000


def _pre_body(x_ref, w_ref, b_ref, dp_ref, h0_ref, g_ref, dis_ref):
    deg = dp_ref[0, :, 0:1] + dp_ref[1, :, 0:1] + 1.0
    dis = lax.rsqrt(deg)
    dis_ref[...] = dis
    h = jnp.dot(x_ref[...], w_ref[...], preferred_element_type=jnp.float32)
    h = jnp.maximum(h + b_ref[...], 0.0)
    h0_ref[...] = h
    g_ref[...] = h * dis


def _dense_pre(x, w_t, b, dp):
    return pl.pallas_call(
        _pre_body,
        grid=(N // _BT,),
        in_specs=[
            pl.BlockSpec((_BT, NFEAT), lambda i: (i, 0)),
            pl.BlockSpec((NFEAT, NHID), lambda i: (0, 0)),
            pl.BlockSpec((1, NHID), lambda i: (0, 0)),
            pl.BlockSpec((NC, _BT, DW), lambda i: (0, i, 0)),
        ],
        out_specs=[
            pl.BlockSpec((_BT, NHID), lambda i: (i, 0)),
            pl.BlockSpec((_BT, NHID), lambda i: (i, 0)),
            pl.BlockSpec((_BT, 1), lambda i: (i, 0)),
        ],
        out_shape=[
            jax.ShapeDtypeStruct((N, NHID), jnp.float32),
            jax.ShapeDtypeStruct((N, NHID), jnp.float32),
            jax.ShapeDtypeStruct((N, 1), jnp.float32),
        ],
    )(x, w_t, b, dp)


def _layer_body(theta, emit_h, p_ref, g_ref, h0_ref, dis_ref, w_ref, *out_refs):
    dis = dis_ref[...]
    s = (1.0 - ALPHA) * (dis * (p_ref[...] + g_ref[...])) + ALPHA * h0_ref[...]
    mm = jnp.dot(s, w_ref[...], preferred_element_type=jnp.float32)
    h = jnp.maximum(theta * mm + (1.0 - theta) * s, 0.0)
    out_refs[0][...] = h * dis
    if emit_h:
        out_refs[1][...] = h


def _dense_layer(theta, emit_h, p, g, h0, dis, w):
    out_specs = [pl.BlockSpec((_BT, NHID), lambda i: (i, 0))]
    out_shape = [jax.ShapeDtypeStruct((N, NHID), jnp.float32)]
    if emit_h:
        out_specs.append(pl.BlockSpec((_BT, NHID), lambda i: (i, 0)))
        out_shape.append(jax.ShapeDtypeStruct((N, NHID), jnp.float32))
    return pl.pallas_call(
        functools.partial(_layer_body, theta, emit_h),
        grid=(N // _BT,),
        in_specs=[
            pl.BlockSpec((_BT, NHID), lambda i: (i, 0)),
            pl.BlockSpec((_BT, NHID), lambda i: (i, 0)),
            pl.BlockSpec((_BT, NHID), lambda i: (i, 0)),
            pl.BlockSpec((_BT, 1), lambda i: (i, 0)),
            pl.BlockSpec((NHID, NHID), lambda i: (0, 0)),
        ],
        out_specs=out_specs,
        out_shape=out_shape,
    )(p, g, h0, dis, w)


def _final_body(h_ref, w_ref, b_ref, o_ref):
    o = jnp.dot(h_ref[...], w_ref[...], preferred_element_type=jnp.float32)
    o_ref[...] = o + b_ref[...]


def _dense_final(h, w_t, b):
    return pl.pallas_call(
        _final_body,
        grid=(N // _BT,),
        in_specs=[
            pl.BlockSpec((_BT, NHID), lambda i: (i, 0)),
            pl.BlockSpec((NHID, NCLASS), lambda i: (0, 0)),
            pl.BlockSpec((1, NCLASS), lambda i: (0, 0)),
        ],
        out_specs=pl.BlockSpec((_BT, NCLASS), lambda i: (i, 0)),
        out_shape=jax.ShapeDtypeStruct((N, NCLASS), jnp.float32),
    )(h, w_t, b)


def kernel(x, edge_idx, fc0_w, fc0_b, convs_w, fc1_w, fc1_b):
    row = edge_idx[0].astype(jnp.int32)
    col = edge_idx[1].astype(jnp.int32)
    self_m = row == col

    row_r = jnp.where(self_m, N, row)
    rowf = jnp.concatenate([row_r, jnp.full((E_PAD - E,), N, jnp.int32)])
    colf = jnp.concatenate([col, jnp.zeros((E_PAD - E,), jnp.int32)])
    rowp = rowf.reshape(NCH, CHUNK)
    colp = colf.reshape(NCH, CHUNK)

    dp = _get_deg_sc()(rowp)
    h0, g, dis = _dense_pre(x, fc0_w.T, fc0_b.reshape(1, NHID), dp)
    h = h0
    for i in range(NLAYERS):
        theta = math.log(LAMDA / (i + 1) + 1.0)
        p = _spmm_sc(g, colp, rowp)
        outs = _dense_layer(theta, i == NLAYERS - 1, p, g, h0, dis, convs_w[i])
        g = outs[0]
        if i == NLAYERS - 1:
            h = outs[1]
    return _dense_final(h, fc1_w.T, fc1_b.reshape(1, NCLASS))


# speedup vs baseline: 3.6846x; 1.3705x over previous
import functools
import math

import jax
import jax.numpy as jnp
from jax import lax
from jax.experimental import pallas as pl
from jax.experimental.pallas import tpu as pltpu
from jax.experimental.pallas import tpu_sc as plsc

N = 10000
E = 320000
NFEAT = 128
NHID = 128
NCLASS = 64
NLAYERS = 8
LAMDA = 0.5
ALPHA = 0.1

NC = 2
NS = 16
FH = NHID // NC
CHUNK = 128
NBUF = 8
CPT = -(-E // (NS * CHUNK * NBUF)) * NBUF
EPT = CPT * CHUNK
E_PAD = EPT * NS
NCH = E_PAD // CHUNK
NGRP = CPT // NBUF
NACC = N + 8
RPT = 632
RPT_LAST_OUT = N - (NS - 1) * RPT
RPT_LAST_ZERO = NACC - (NS - 1) * RPT
LANES = 16


def _spmm_sc_body(
    gt_hbm, col_hbm, row_hbm, out_hbm, cidx_v, ridx_v, rows_v, zbuf_v, tab_sh, acc_sh, gsems
):
    c = lax.axis_index("c")
    s = lax.axis_index("s")

    zeros = jnp.zeros((2 * LANES,), jnp.bfloat16)

    def zero_row(i, _):
        for j in range(FH // (2 * LANES)):
            zbuf_v[i, pl.ds(j * 2 * LANES, 2 * LANES)] = zeros
        return 0

    lax.fori_loop(0, CHUNK, zero_row, 0)

    def prep_slab(base, nrows, nzero):
        nfull = nzero // CHUNK
        for k in range(nfull):
            pltpu.sync_copy(zbuf_v, acc_sh.at[pl.ds(base + k * CHUNK, CHUNK)])
        rem = nzero - nfull * CHUNK
        if rem:
            pltpu.sync_copy(
                zbuf_v.at[pl.ds(0, rem)], acc_sh.at[pl.ds(base + nfull * CHUNK, rem)]
            )
        pltpu.sync_copy(
            gt_hbm.at[pl.ds(base, nrows), pl.ds(c * FH, FH)], tab_sh.at[pl.ds(base, nrows)]
        )

    @pl.when(s < NS - 1)
    def _():
        prep_slab(s * RPT, RPT, RPT)

    @pl.when(s == NS - 1)
    def _():
        prep_slab((NS - 1) * RPT, RPT_LAST_OUT, RPT_LAST_ZERO)

    plsc.subcore_barrier()

    cbase = s * CPT

    def body(g, _):
        base = cbase + g * NBUF
        pltpu.sync_copy(col_hbm.at[pl.ds(base, NBUF)], cidx_v)
        pltpu.sync_copy(row_hbm.at[pl.ds(base, NBUF)], ridx_v)
        gd = [
            pltpu.async_copy(tab_sh.at[cidx_v.at[b]], rows_v.at[b], gsems[b])
            for b in range(NBUF)
        ]
        for b in range(NBUF):
            gd[b].wait()
            pltpu.sync_copy(rows_v.at[b], acc_sh.at[ridx_v.at[b]], add=True)
        return 0

    lax.fori_loop(0, NGRP, body, 0)

    plsc.subcore_barrier()

    @pl.when(s < NS - 1)
    def _():
        ob = s * RPT
        pltpu.sync_copy(
            acc_sh.at[pl.ds(ob, RPT)], out_hbm.at[pl.ds(ob, RPT), pl.ds(c * FH, FH)]
        )

    @pl.when(s == NS - 1)
    def _():
        ob = (NS - 1) * RPT
        pltpu.sync_copy(
            acc_sh.at[pl.ds(ob, RPT_LAST_OUT)],
            out_hbm.at[pl.ds(ob, RPT_LAST_OUT), pl.ds(c * FH, FH)],
        )


@functools.cache
def _get_spmm_sc():
    return pl.kernel(
        _spmm_sc_body,
        out_type=jax.ShapeDtypeStruct((N, NHID), jnp.bfloat16),
        mesh=plsc.VectorSubcoreMesh(
            core_axis_name="c", subcore_axis_name="s", num_cores=NC, num_subcores=NS
        ),
        scratch_types=[
            pltpu.VMEM((NBUF, CHUNK), jnp.int32),
            pltpu.VMEM((NBUF, CHUNK), jnp.int32),
            pltpu.VMEM((NBUF, CHUNK, FH), jnp.bfloat16),
            pltpu.VMEM((CHUNK, FH), jnp.bfloat16),
            pltpu.VMEM_SHARED((N, FH), jnp.bfloat16),
            pltpu.VMEM_SHARED((NACC, FH), jnp.bfloat16),
            [pltpu.SemaphoreType.DMA] * NBUF,
        ],
        compiler_params=pltpu.CompilerParams(use_tc_tiling_on_sc=False),
    )


def _spmm_sc(gt, colp, rowp):
    return _get_spmm_sc()(gt, colp, rowp)


DW = LANES
CPT_D = NCH // NC // NS


def _deg_sc_body(row_hbm, out_hbm, ridx_v, ones_v, zbuf_v, acc_sh):
    c = lax.axis_index("c")
    s = lax.axis_index("s")

    ones = jnp.full((LANES,), 1.0, jnp.float32)
    zeros = jnp.zeros((LANES,), jnp.float32)

    def fill_row(i, _):
        ones_v[i, pl.ds(0, LANES)] = ones
        zbuf_v[i, pl.ds(0, LANES)] = zeros
        return 0

    lax.fori_loop(0, CHUNK, fill_row, 0)

    def zero_slab(base, nzero):
        nfull = nzero // CHUNK
        for k in range(nfull):
            pltpu.sync_copy(zbuf_v, acc_sh.at[pl.ds(base + k * CHUNK, CHUNK)])
        rem = nzero - nfull * CHUNK
        if rem:
            pltpu.sync_copy(
                zbuf_v.at[pl.ds(0, rem)], acc_sh.at[pl.ds(base + nfull * CHUNK, rem)]
            )

    @pl.when(s < NS - 1)
    def _():
        zero_slab(s * RPT, RPT)

    @pl.when(s == NS - 1)
    def _():
        zero_slab((NS - 1) * RPT, RPT_LAST_ZERO)

    plsc.subcore_barrier()

    cbase = (c * NS + s) * CPT_D
    pltpu.sync_copy(row_hbm.at[pl.ds(cbase, CPT_D)], ridx_v)

    def body(k, _):
        pltpu.sync_copy(ones_v, acc_sh.at[ridx_v.at[k]], add=True)
        return 0

    lax.fori_loop(0, CPT_D, body, 0)

    plsc.subcore_barrier()

    @pl.when(s < NS - 1)
    def _():
        ob = s * RPT
        pltpu.sync_copy(acc_sh.at[pl.ds(ob, RPT)], out_hbm.at[c, pl.ds(ob, RPT)])

    @pl.when(s == NS - 1)
    def _():
        ob = (NS - 1) * RPT
        pltpu.sync_copy(
            acc_sh.at[pl.ds(ob, RPT_LAST_OUT)], out_hbm.at[c, pl.ds(ob, RPT_LAST_OUT)]
        )


@functools.cache
def _get_deg_sc():
    return pl.kernel(
        _deg_sc_body,
        out_type=jax.ShapeDtypeStruct((NC, N, DW), jnp.float32),
        mesh=plsc.VectorSubcoreMesh(
            core_axis_name="c", subcore_axis_name="s", num_cores=NC, num_subcores=NS
        ),
        scratch_types=[
            pltpu.VMEM((CPT_D, CHUNK), jnp.int32),
            pltpu.VMEM((CHUNK, DW), jnp.float32),
            pltpu.VMEM((CHUNK, DW), jnp.float32),
            pltpu.VMEM_SHARED((NACC, DW), jnp.float32),
        ],
        compiler_params=pltpu.CompilerParams(use_tc_tiling_on_sc=False),
    )


_BT = 1000


def _pre_body(x_ref, w_ref, b_ref, dp_ref, h0_ref, g_ref, dis_ref):
    deg = dp_ref[0, :, 0:1] + dp_ref[1, :, 0:1] + 1.0
    dis = lax.rsqrt(deg)
    dis_ref[...] = dis
    h = jnp.dot(x_ref[...], w_ref[...], preferred_element_type=jnp.float32)
    h = jnp.maximum(h + b_ref[...], 0.0)
    h0_ref[...] = h
    g_ref[...] = (h * dis).astype(jnp.bfloat16)


def _dense_pre(x, w_t, b, dp):
    return pl.pallas_call(
        _pre_body,
        grid=(N // _BT,),
        in_specs=[
            pl.BlockSpec((_BT, NFEAT), lambda i: (i, 0)),
            pl.BlockSpec((NFEAT, NHID), lambda i: (0, 0)),
            pl.BlockSpec((1, NHID), lambda i: (0, 0)),
            pl.BlockSpec((NC, _BT, DW), lambda i: (0, i, 0)),
        ],
        out_specs=[
            pl.BlockSpec((_BT, NHID), lambda i: (i, 0)),
            pl.BlockSpec((_BT, NHID), lambda i: (i, 0)),
            pl.BlockSpec((_BT, 1), lambda i: (i, 0)),
        ],
        out_shape=[
            jax.ShapeDtypeStruct((N, NHID), jnp.float32),
            jax.ShapeDtypeStruct((N, NHID), jnp.bfloat16),
            jax.ShapeDtypeStruct((N, 1), jnp.float32),
        ],
    )(x, w_t, b, dp)


def _layer_body(theta, emit_h, p_ref, g_ref, h0_ref, dis_ref, w_ref, *out_refs):
    dis = dis_ref[...]
    msgs = p_ref[...].astype(jnp.float32) + g_ref[...].astype(jnp.float32)
    s = (1.0 - ALPHA) * (dis * msgs) + ALPHA * h0_ref[...]
    mm = jnp.dot(s, w_ref[...], preferred_element_type=jnp.float32)
    h = jnp.maximum(theta * mm + (1.0 - theta) * s, 0.0)
    out_refs[0][...] = (h * dis).astype(jnp.bfloat16)
    if emit_h:
        out_refs[1][...] = h


def _dense_layer(theta, emit_h, p, g, h0, dis, w):
    out_specs = [pl.BlockSpec((_BT, NHID), lambda i: (i, 0))]
    out_shape = [jax.ShapeDtypeStruct((N, NHID), jnp.bfloat16)]
    if emit_h:
        out_specs.append(pl.BlockSpec((_BT, NHID), lambda i: (i, 0)))
        out_shape.append(jax.ShapeDtypeStruct((N, NHID), jnp.float32))
    return pl.pallas_call(
        functools.partial(_layer_body, theta, emit_h),
        grid=(N // _BT,),
        in_specs=[
            pl.BlockSpec((_BT, NHID), lambda i: (i, 0)),
            pl.BlockSpec((_BT, NHID), lambda i: (i, 0)),
            pl.BlockSpec((_BT, NHID), lambda i: (i, 0)),
            pl.BlockSpec((_BT, 1), lambda i: (i, 0)),
            pl.BlockSpec((NHID, NHID), lambda i: (0, 0)),
        ],
        out_specs=out_specs,
        out_shape=out_shape,
    )(p, g, h0, dis, w)


def _final_body(h_ref, w_ref, b_ref, o_ref):
    o = jnp.dot(h_ref[...], w_ref[...], preferred_element_type=jnp.float32)
    o_ref[...] = o + b_ref[...]


def _dense_final(h, w_t, b):
    return pl.pallas_call(
        _final_body,
        grid=(N // _BT,),
        in_specs=[
            pl.BlockSpec((_BT, NHID), lambda i: (i, 0)),
            pl.BlockSpec((NHID, NCLASS), lambda i: (0, 0)),
            pl.BlockSpec((1, NCLASS), lambda i: (0, 0)),
        ],
        out_specs=pl.BlockSpec((_BT, NCLASS), lambda i: (i, 0)),
        out_shape=jax.ShapeDtypeStruct((N, NCLASS), jnp.float32),
    )(h, w_t, b)


def kernel(x, edge_idx, fc0_w, fc0_b, convs_w, fc1_w, fc1_b):
    row = edge_idx[0].astype(jnp.int32)
    col = edge_idx[1].astype(jnp.int32)
    self_m = row == col

    row_r = jnp.where(self_m, N, row)
    rowf = jnp.concatenate([row_r, jnp.full((E_PAD - E,), N, jnp.int32)])
    colf = jnp.concatenate([col, jnp.zeros((E_PAD - E,), jnp.int32)])
    rowp = rowf.reshape(NCH, CHUNK)
    colp = colf.reshape(NCH, CHUNK)

    dp = _get_deg_sc()(rowp)
    h0, g, dis = _dense_pre(x, fc0_w.T, fc0_b.reshape(1, NHID), dp)
    h = h0
    for i in range(NLAYERS):
        theta = math.log(LAMDA / (i + 1) + 1.0)
        p = _spmm_sc(g, colp, rowp)
        outs = _dense_layer(theta, i == NLAYERS - 1, p, g, h0, dis, convs_w[i])
        g = outs[0]
        if i == NLAYERS - 1:
            h = outs[1]
    return _dense_final(h, fc1_w.T, fc1_b.reshape(1, NCLASS))


# speedup vs baseline: 3.7935x; 1.0295x over previous
import functools
import math

import jax
import jax.numpy as jnp
from jax import lax
from jax.experimental import pallas as pl
from jax.experimental.pallas import tpu as pltpu
from jax.experimental.pallas import tpu_sc as plsc

N = 10000
E = 320000
NFEAT = 128
NHID = 128
NCLASS = 64
NLAYERS = 8
LAMDA = 0.5
ALPHA = 0.1

NC = 2
NS = 16
FH = NHID // NC
CHUNK = 128
NBUF = 8
CPT = -(-E // (NS * CHUNK * NBUF)) * NBUF
EPT = CPT * CHUNK
E_PAD = EPT * NS
NCH = E_PAD // CHUNK
NGRP = CPT // NBUF
NACC = N + 8
RPT = 632
RPT_LAST_OUT = N - (NS - 1) * RPT
RPT_LAST_ZERO = NACC - (NS - 1) * RPT
LANES = 16


def _spmm_sc_body(
    gt_hbm, col_hbm, row_hbm, out_hbm, cidx_v, ridx_v, rows_v, zbuf_v, tab_sh, acc_sh, gsems, ssems
):
    c = lax.axis_index("c")
    s = lax.axis_index("s")

    zeros = jnp.zeros((2 * LANES,), jnp.bfloat16)

    def zero_row(i, _):
        for j in range(FH // (2 * LANES)):
            zbuf_v[i, pl.ds(j * 2 * LANES, 2 * LANES)] = zeros
        return 0

    lax.fori_loop(0, CHUNK, zero_row, 0)

    def prep_slab(base, nrows, nzero):
        nfull = nzero // CHUNK
        for k in range(nfull):
            pltpu.sync_copy(zbuf_v, acc_sh.at[pl.ds(base + k * CHUNK, CHUNK)])
        rem = nzero - nfull * CHUNK
        if rem:
            pltpu.sync_copy(
                zbuf_v.at[pl.ds(0, rem)], acc_sh.at[pl.ds(base + nfull * CHUNK, rem)]
            )
        pltpu.sync_copy(
            gt_hbm.at[pl.ds(base, nrows), pl.ds(c * FH, FH)], tab_sh.at[pl.ds(base, nrows)]
        )

    @pl.when(s < NS - 1)
    def _():
        prep_slab(s * RPT, RPT, RPT)

    @pl.when(s == NS - 1)
    def _():
        prep_slab((NS - 1) * RPT, RPT_LAST_OUT, RPT_LAST_ZERO)

    plsc.subcore_barrier()

    cbase = s * CPT

    def body(g, _):
        base = cbase + g * NBUF
        pltpu.sync_copy(col_hbm.at[pl.ds(base, NBUF)], cidx_v)
        pltpu.sync_copy(row_hbm.at[pl.ds(base, NBUF)], ridx_v)
        gd = [
            pltpu.async_copy(tab_sh.at[cidx_v.at[b]], rows_v.at[b], gsems[b])
            for b in range(NBUF)
        ]
        sd = []
        for b in range(NBUF):
            gd[b].wait()
            sd.append(
                pltpu.async_copy(rows_v.at[b], acc_sh.at[ridx_v.at[b]], ssems[b], add=True)
            )
        for b in range(NBUF):
            sd[b].wait()
        return 0

    lax.fori_loop(0, NGRP, body, 0)

    plsc.subcore_barrier()

    @pl.when(s < NS - 1)
    def _():
        ob = s * RPT
        pltpu.sync_copy(
            acc_sh.at[pl.ds(ob, RPT)], out_hbm.at[pl.ds(ob, RPT), pl.ds(c * FH, FH)]
        )

    @pl.when(s == NS - 1)
    def _():
        ob = (NS - 1) * RPT
        pltpu.sync_copy(
            acc_sh.at[pl.ds(ob, RPT_LAST_OUT)],
            out_hbm.at[pl.ds(ob, RPT_LAST_OUT), pl.ds(c * FH, FH)],
        )


@functools.cache
def _get_spmm_sc():
    return pl.kernel(
        _spmm_sc_body,
        out_type=jax.ShapeDtypeStruct((N, NHID), jnp.bfloat16),
        mesh=plsc.VectorSubcoreMesh(
            core_axis_name="c", subcore_axis_name="s", num_cores=NC, num_subcores=NS
        ),
        scratch_types=[
            pltpu.VMEM((NBUF, CHUNK), jnp.int32),
            pltpu.VMEM((NBUF, CHUNK), jnp.int32),
            pltpu.VMEM((NBUF, CHUNK, FH), jnp.bfloat16),
            pltpu.VMEM((CHUNK, FH), jnp.bfloat16),
            pltpu.VMEM_SHARED((N, FH), jnp.bfloat16),
            pltpu.VMEM_SHARED((NACC, FH), jnp.bfloat16),
            [pltpu.SemaphoreType.DMA] * NBUF,
            [pltpu.SemaphoreType.DMA] * NBUF,
        ],
        compiler_params=pltpu.CompilerParams(use_tc_tiling_on_sc=False),
    )


def _spmm_sc(gt, colp, rowp):
    return _get_spmm_sc()(gt, colp, rowp)


DW = LANES
CPT_D = NCH // NC // NS


def _deg_sc_body(row_hbm, out_hbm, ridx_v, ones_v, zbuf_v, acc_sh):
    c = lax.axis_index("c")
    s = lax.axis_index("s")

    ones = jnp.full((LANES,), 1.0, jnp.float32)
    zeros = jnp.zeros((LANES,), jnp.float32)

    def fill_row(i, _):
        ones_v[i, pl.ds(0, LANES)] = ones
        zbuf_v[i, pl.ds(0, LANES)] = zeros
        return 0

    lax.fori_loop(0, CHUNK, fill_row, 0)

    def zero_slab(base, nzero):
        nfull = nzero // CHUNK
        for k in range(nfull):
            pltpu.sync_copy(zbuf_v, acc_sh.at[pl.ds(base + k * CHUNK, CHUNK)])
        rem = nzero - nfull * CHUNK
        if rem:
            pltpu.sync_copy(
                zbuf_v.at[pl.ds(0, rem)], acc_sh.at[pl.ds(base + nfull * CHUNK, rem)]
            )

    @pl.when(s < NS - 1)
    def _():
        zero_slab(s * RPT, RPT)

    @pl.when(s == NS - 1)
    def _():
        zero_slab((NS - 1) * RPT, RPT_LAST_ZERO)

    plsc.subcore_barrier()

    cbase = (c * NS + s) * CPT_D
    pltpu.sync_copy(row_hbm.at[pl.ds(cbase, CPT_D)], ridx_v)

    def body(k, _):
        pltpu.sync_copy(ones_v, acc_sh.at[ridx_v.at[k]], add=True)
        return 0

    lax.fori_loop(0, CPT_D, body, 0)

    plsc.subcore_barrier()

    @pl.when(s < NS - 1)
    def _():
        ob = s * RPT
        pltpu.sync_copy(acc_sh.at[pl.ds(ob, RPT)], out_hbm.at[c, pl.ds(ob, RPT)])

    @pl.when(s == NS - 1)
    def _():
        ob = (NS - 1) * RPT
        pltpu.sync_copy(
            acc_sh.at[pl.ds(ob, RPT_LAST_OUT)], out_hbm.at[c, pl.ds(ob, RPT_LAST_OUT)]
        )


@functools.cache
def _get_deg_sc():
    return pl.kernel(
        _deg_sc_body,
        out_type=jax.ShapeDtypeStruct((NC, N, DW), jnp.float32),
        mesh=plsc.VectorSubcoreMesh(
            core_axis_name="c", subcore_axis_name="s", num_cores=NC, num_subcores=NS
        ),
        scratch_types=[
            pltpu.VMEM((CPT_D, CHUNK), jnp.int32),
            pltpu.VMEM((CHUNK, DW), jnp.float32),
            pltpu.VMEM((CHUNK, DW), jnp.float32),
            pltpu.VMEM_SHARED((NACC, DW), jnp.float32),
        ],
        compiler_params=pltpu.CompilerParams(use_tc_tiling_on_sc=False),
    )


_BT = 1000


def _pre_body(x_ref, w_ref, b_ref, dp_ref, h0_ref, g_ref, dis_ref):
    deg = dp_ref[0, :, 0:1] + dp_ref[1, :, 0:1] + 1.0
    dis = lax.rsqrt(deg)
    dis_ref[...] = dis
    h = jnp.dot(x_ref[...], w_ref[...], preferred_element_type=jnp.float32)
    h = jnp.maximum(h + b_ref[...], 0.0)
    h0_ref[...] = h
    g_ref[...] = (h * dis).astype(jnp.bfloat16)


def _dense_pre(x, w_t, b, dp):
    return pl.pallas_call(
        _pre_body,
        grid=(N // _BT,),
        in_specs=[
            pl.BlockSpec((_BT, NFEAT), lambda i: (i, 0)),
            pl.BlockSpec((NFEAT, NHID), lambda i: (0, 0)),
            pl.BlockSpec((1, NHID), lambda i: (0, 0)),
            pl.BlockSpec((NC, _BT, DW), lambda i: (0, i, 0)),
        ],
        out_specs=[
            pl.BlockSpec((_BT, NHID), lambda i: (i, 0)),
            pl.BlockSpec((_BT, NHID), lambda i: (i, 0)),
            pl.BlockSpec((_BT, 1), lambda i: (i, 0)),
        ],
        out_shape=[
            jax.ShapeDtypeStruct((N, NHID), jnp.float32),
            jax.ShapeDtypeStruct((N, NHID), jnp.bfloat16),
            jax.ShapeDtypeStruct((N, 1), jnp.float32),
        ],
    )(x, w_t, b, dp)


def _layer_body(theta, emit_h, p_ref, g_ref, h0_ref, dis_ref, w_ref, *out_refs):
    dis = dis_ref[...]
    msgs = p_ref[...].astype(jnp.float32) + g_ref[...].astype(jnp.float32)
    s = (1.0 - ALPHA) * (dis * msgs) + ALPHA * h0_ref[...]
    mm = jnp.dot(s, w_ref[...], preferred_element_type=jnp.float32)
    h = jnp.maximum(theta * mm + (1.0 - theta) * s, 0.0)
    out_refs[0][...] = (h * dis).astype(jnp.bfloat16)
    if emit_h:
        out_refs[1][...] = h


def _dense_layer(theta, emit_h, p, g, h0, dis, w):
    out_specs = [pl.BlockSpec((_BT, NHID), lambda i: (i, 0))]
    out_shape = [jax.ShapeDtypeStruct((N, NHID), jnp.bfloat16)]
    if emit_h:
        out_specs.append(pl.BlockSpec((_BT, NHID), lambda i: (i, 0)))
        out_shape.append(jax.ShapeDtypeStruct((N, NHID), jnp.float32))
    return pl.pallas_call(
        functools.partial(_layer_body, theta, emit_h),
        grid=(N // _BT,),
        in_specs=[
            pl.BlockSpec((_BT, NHID), lambda i: (i, 0)),
            pl.BlockSpec((_BT, NHID), lambda i: (i, 0)),
            pl.BlockSpec((_BT, NHID), lambda i: (i, 0)),
            pl.BlockSpec((_BT, 1), lambda i: (i, 0)),
            pl.BlockSpec((NHID, NHID), lambda i: (0, 0)),
        ],
        out_specs=out_specs,
        out_shape=out_shape,
    )(p, g, h0, dis, w)


def _final_body(h_ref, w_ref, b_ref, o_ref):
    o = jnp.dot(h_ref[...], w_ref[...], preferred_element_type=jnp.float32)
    o_ref[...] = o + b_ref[...]


def _dense_final(h, w_t, b):
    return pl.pallas_call(
        _final_body,
        grid=(N // _BT,),
        in_specs=[
            pl.BlockSpec((_BT, NHID), lambda i: (i, 0)),
            pl.BlockSpec((NHID, NCLASS), lambda i: (0, 0)),
            pl.BlockSpec((1, NCLASS), lambda i: (0, 0)),
        ],
        out_specs=pl.BlockSpec((_BT, NCLASS), lambda i: (i, 0)),
        out_shape=jax.ShapeDtypeStruct((N, NCLASS), jnp.float32),
    )(h, w_t, b)


def kernel(x, edge_idx, fc0_w, fc0_b, convs_w, fc1_w, fc1_b):
    row = edge_idx[0].astype(jnp.int32)
    col = edge_idx[1].astype(jnp.int32)
    self_m = row == col

    row_r = jnp.where(self_m, N, row)
    rowf = jnp.concatenate([row_r, jnp.full((E_PAD - E,), N, jnp.int32)])
    colf = jnp.concatenate([col, jnp.zeros((E_PAD - E,), jnp.int32)])
    rowp = rowf.reshape(NCH, CHUNK)
    colp = colf.reshape(NCH, CHUNK)

    dp = _get_deg_sc()(rowp)
    h0, g, dis = _dense_pre(x, fc0_w.T, fc0_b.reshape(1, NHID), dp)
    h = h0
    for i in range(NLAYERS):
        theta = math.log(LAMDA / (i + 1) + 1.0)
        p = _spmm_sc(g, colp, rowp)
        outs = _dense_layer(theta, i == NLAYERS - 1, p, g, h0, dis, convs_w[i])
        g = outs[0]
        if i == NLAYERS - 1:
            h = outs[1]
    return _dense_final(h, fc1_w.T, fc1_b.reshape(1, NCLASS))


# speedup vs baseline: 3.8883x; 1.0250x over previous
import functools
import math

import jax
import jax.numpy as jnp
from jax import lax
from jax.experimental import pallas as pl
from jax.experimental.pallas import tpu as pltpu
from jax.experimental.pallas import tpu_sc as plsc

N = 10000
E = 320000
NFEAT = 128
NHID = 128
NCLASS = 64
NLAYERS = 8
LAMDA = 0.5
ALPHA = 0.1

NC = 2
NS = 16
FH = NHID // NC
CHUNK = 128
NBUF = 8
CPT = -(-E // (NS * CHUNK * NBUF)) * NBUF
EPT = CPT * CHUNK
E_PAD = EPT * NS
NCH = E_PAD // CHUNK
NGRP = CPT // NBUF
NACC = N + 8
RPT = 632
RPT_LAST_OUT = N - (NS - 1) * RPT
RPT_LAST_ZERO = NACC - (NS - 1) * RPT
LANES = 16


def _spmm_sc_body(
    gt_hbm, col_hbm, row_hbm, out_hbm, cidx_v, ridx_v, rows_v, zbuf_v, tab_sh, acc_sh, gsems, ssems
):
    c = lax.axis_index("c")
    s = lax.axis_index("s")

    zeros = jnp.zeros((2 * LANES,), jnp.bfloat16)

    def zero_row(i, _):
        for j in range(FH // (2 * LANES)):
            zbuf_v[i, pl.ds(j * 2 * LANES, 2 * LANES)] = zeros
        return 0

    lax.fori_loop(0, CHUNK, zero_row, 0)

    def prep_slab(base, nrows, nzero):
        nfull = nzero // CHUNK
        for k in range(nfull):
            pltpu.sync_copy(zbuf_v, acc_sh.at[pl.ds(base + k * CHUNK, CHUNK)])
        rem = nzero - nfull * CHUNK
        if rem:
            pltpu.sync_copy(
                zbuf_v.at[pl.ds(0, rem)], acc_sh.at[pl.ds(base + nfull * CHUNK, rem)]
            )
        pltpu.sync_copy(
            gt_hbm.at[pl.ds(base, nrows), pl.ds(c * FH, FH)], tab_sh.at[pl.ds(base, nrows)]
        )

    @pl.when(s < NS - 1)
    def _():
        prep_slab(s * RPT, RPT, RPT)

    @pl.when(s == NS - 1)
    def _():
        prep_slab((NS - 1) * RPT, RPT_LAST_OUT, RPT_LAST_ZERO)

    plsc.subcore_barrier()

    cbase = s * CPT

    def body(g, _):
        base = cbase + g * NBUF
        pltpu.sync_copy(col_hbm.at[pl.ds(base, NBUF)], cidx_v)
        pltpu.sync_copy(row_hbm.at[pl.ds(base, NBUF)], ridx_v)
        gd = [
            pltpu.async_copy(tab_sh.at[cidx_v.at[b]], rows_v.at[b], gsems[b])
            for b in range(NBUF)
        ]
        sd = []
        for b in range(NBUF):
            gd[b].wait()
            sd.append(
                pltpu.async_copy(rows_v.at[b], acc_sh.at[ridx_v.at[b]], ssems[b], add=True)
            )
        for b in range(NBUF):
            sd[b].wait()
        return 0

    lax.fori_loop(0, NGRP, body, 0)

    plsc.subcore_barrier()

    @pl.when(s < NS - 1)
    def _():
        ob = s * RPT
        pltpu.sync_copy(
            acc_sh.at[pl.ds(ob, RPT)], out_hbm.at[pl.ds(ob, RPT), pl.ds(c * FH, FH)]
        )

    @pl.when(s == NS - 1)
    def _():
        ob = (NS - 1) * RPT
        pltpu.sync_copy(
            acc_sh.at[pl.ds(ob, RPT_LAST_OUT)],
            out_hbm.at[pl.ds(ob, RPT_LAST_OUT), pl.ds(c * FH, FH)],
        )


@functools.cache
def _get_spmm_sc():
    return pl.kernel(
        _spmm_sc_body,
        out_type=jax.ShapeDtypeStruct((N, NHID), jnp.bfloat16),
        mesh=plsc.VectorSubcoreMesh(
            core_axis_name="c", subcore_axis_name="s", num_cores=NC, num_subcores=NS
        ),
        scratch_types=[
            pltpu.VMEM((NBUF, CHUNK), jnp.int32),
            pltpu.VMEM((NBUF, CHUNK), jnp.int32),
            pltpu.VMEM((NBUF, CHUNK, FH), jnp.bfloat16),
            pltpu.VMEM((CHUNK, FH), jnp.bfloat16),
            pltpu.VMEM_SHARED((N, FH), jnp.bfloat16),
            pltpu.VMEM_SHARED((NACC, FH), jnp.bfloat16),
            [pltpu.SemaphoreType.DMA] * NBUF,
            [pltpu.SemaphoreType.DMA] * NBUF,
        ],
        compiler_params=pltpu.CompilerParams(use_tc_tiling_on_sc=False),
    )


def _spmm_sc(gt, colp, rowp):
    return _get_spmm_sc()(gt, colp, rowp)


DW = LANES
CPT_D = NCH // NC // NS


def _deg_sc_body(row_hbm, out_hbm, ridx_v, ones_v, zbuf_v, acc_sh):
    c = lax.axis_index("c")
    s = lax.axis_index("s")

    ones = jnp.full((LANES,), 1.0, jnp.float32)
    zeros = jnp.zeros((LANES,), jnp.float32)

    def fill_row(i, _):
        ones_v[i, pl.ds(0, LANES)] = ones
        zbuf_v[i, pl.ds(0, LANES)] = zeros
        return 0

    lax.fori_loop(0, CHUNK, fill_row, 0)

    def zero_slab(base, nzero):
        nfull = nzero // CHUNK
        for k in range(nfull):
            pltpu.sync_copy(zbuf_v, acc_sh.at[pl.ds(base + k * CHUNK, CHUNK)])
        rem = nzero - nfull * CHUNK
        if rem:
            pltpu.sync_copy(
                zbuf_v.at[pl.ds(0, rem)], acc_sh.at[pl.ds(base + nfull * CHUNK, rem)]
            )

    @pl.when(s < NS - 1)
    def _():
        zero_slab(s * RPT, RPT)

    @pl.when(s == NS - 1)
    def _():
        zero_slab((NS - 1) * RPT, RPT_LAST_ZERO)

    plsc.subcore_barrier()

    cbase = (c * NS + s) * CPT_D
    pltpu.sync_copy(row_hbm.at[pl.ds(cbase, CPT_D)], ridx_v)

    def body(k, _):
        pltpu.sync_copy(ones_v, acc_sh.at[ridx_v.at[k]], add=True)
        return 0

    lax.fori_loop(0, CPT_D, body, 0)

    plsc.subcore_barrier()

    @pl.when(s < NS - 1)
    def _():
        ob = s * RPT
        pltpu.sync_copy(acc_sh.at[pl.ds(ob, RPT)], out_hbm.at[c, pl.ds(ob, RPT)])

    @pl.when(s == NS - 1)
    def _():
        ob = (NS - 1) * RPT
        pltpu.sync_copy(
            acc_sh.at[pl.ds(ob, RPT_LAST_OUT)], out_hbm.at[c, pl.ds(ob, RPT_LAST_OUT)]
        )


@functools.cache
def _get_deg_sc():
    return pl.kernel(
        _deg_sc_body,
        out_type=jax.ShapeDtypeStruct((NC, N, DW), jnp.float32),
        mesh=plsc.VectorSubcoreMesh(
            core_axis_name="c", subcore_axis_name="s", num_cores=NC, num_subcores=NS
        ),
        scratch_types=[
            pltpu.VMEM((CPT_D, CHUNK), jnp.int32),
            pltpu.VMEM((CHUNK, DW), jnp.float32),
            pltpu.VMEM((CHUNK, DW), jnp.float32),
            pltpu.VMEM_SHARED((NACC, DW), jnp.float32),
        ],
        compiler_params=pltpu.CompilerParams(use_tc_tiling_on_sc=False),
    )


_BT = 2000


def _fc0_body(x_ref, w_ref, b_ref, h0_ref):
    h = jnp.dot(x_ref[...], w_ref[...], preferred_element_type=jnp.float32)
    h0_ref[...] = jnp.maximum(h + b_ref[...], 0.0)


def _dense_fc0(x, w_t, b):
    return pl.pallas_call(
        _fc0_body,
        grid=(N // _BT,),
        in_specs=[
            pl.BlockSpec((_BT, NFEAT), lambda i: (i, 0)),
            pl.BlockSpec((NFEAT, NHID), lambda i: (0, 0)),
            pl.BlockSpec((1, NHID), lambda i: (0, 0)),
        ],
        out_specs=pl.BlockSpec((_BT, NHID), lambda i: (i, 0)),
        out_shape=jax.ShapeDtypeStruct((N, NHID), jnp.float32),
    )(x, w_t, b)


def _pre_body(h0_ref, dp_ref, g_ref, dis_ref):
    deg = dp_ref[0, :, 0:1] + dp_ref[1, :, 0:1] + 1.0
    dis = lax.rsqrt(deg)
    dis_ref[...] = dis
    g_ref[...] = (h0_ref[...] * dis).astype(jnp.bfloat16)


def _dense_pre(h0, dp):
    return pl.pallas_call(
        _pre_body,
        grid=(N // _BT,),
        in_specs=[
            pl.BlockSpec((_BT, NHID), lambda i: (i, 0)),
            pl.BlockSpec((NC, _BT, DW), lambda i: (0, i, 0)),
        ],
        out_specs=[
            pl.BlockSpec((_BT, NHID), lambda i: (i, 0)),
            pl.BlockSpec((_BT, 1), lambda i: (i, 0)),
        ],
        out_shape=[
            jax.ShapeDtypeStruct((N, NHID), jnp.bfloat16),
            jax.ShapeDtypeStruct((N, 1), jnp.float32),
        ],
    )(h0, dp)


def _layer_body(theta, emit_h, p_ref, g_ref, h0_ref, dis_ref, w_ref, *out_refs):
    dis = dis_ref[...]
    msgs = p_ref[...].astype(jnp.float32) + g_ref[...].astype(jnp.float32)
    s = (1.0 - ALPHA) * (dis * msgs) + ALPHA * h0_ref[...]
    mm = jnp.dot(s, w_ref[...], preferred_element_type=jnp.float32)
    h = jnp.maximum(theta * mm + (1.0 - theta) * s, 0.0)
    out_refs[0][...] = (h * dis).astype(jnp.bfloat16)
    if emit_h:
        out_refs[1][...] = h


def _dense_layer(theta, emit_h, p, g, h0, dis, w):
    out_specs = [pl.BlockSpec((_BT, NHID), lambda i: (i, 0))]
    out_shape = [jax.ShapeDtypeStruct((N, NHID), jnp.bfloat16)]
    if emit_h:
        out_specs.append(pl.BlockSpec((_BT, NHID), lambda i: (i, 0)))
        out_shape.append(jax.ShapeDtypeStruct((N, NHID), jnp.float32))
    return pl.pallas_call(
        functools.partial(_layer_body, theta, emit_h),
        grid=(N // _BT,),
        in_specs=[
            pl.BlockSpec((_BT, NHID), lambda i: (i, 0)),
            pl.BlockSpec((_BT, NHID), lambda i: (i, 0)),
            pl.BlockSpec((_BT, NHID), lambda i: (i, 0)),
            pl.BlockSpec((_BT, 1), lambda i: (i, 0)),
            pl.BlockSpec((NHID, NHID), lambda i: (0, 0)),
        ],
        out_specs=out_specs,
        out_shape=out_shape,
    )(p, g, h0, dis, w)


def _final_body(h_ref, w_ref, b_ref, o_ref):
    o = jnp.dot(h_ref[...], w_ref[...], preferred_element_type=jnp.float32)
    o_ref[...] = o + b_ref[...]


def _dense_final(h, w_t, b):
    return pl.pallas_call(
        _final_body,
        grid=(N // _BT,),
        in_specs=[
            pl.BlockSpec((_BT, NHID), lambda i: (i, 0)),
            pl.BlockSpec((NHID, NCLASS), lambda i: (0, 0)),
            pl.BlockSpec((1, NCLASS), lambda i: (0, 0)),
        ],
        out_specs=pl.BlockSpec((_BT, NCLASS), lambda i: (i, 0)),
        out_shape=jax.ShapeDtypeStruct((N, NCLASS), jnp.float32),
    )(h, w_t, b)


def kernel(x, edge_idx, fc0_w, fc0_b, convs_w, fc1_w, fc1_b):
    row = edge_idx[0].astype(jnp.int32)
    col = edge_idx[1].astype(jnp.int32)
    self_m = row == col

    row_r = jnp.where(self_m, N, row)
    rowf = jnp.concatenate([row_r, jnp.full((E_PAD - E,), N, jnp.int32)])
    colf = jnp.concatenate([col, jnp.zeros((E_PAD - E,), jnp.int32)])
    rowp = rowf.reshape(NCH, CHUNK)
    colp = colf.reshape(NCH, CHUNK)

    dp = _get_deg_sc()(rowp)
    h0 = _dense_fc0(x, fc0_w.T, fc0_b.reshape(1, NHID))
    g, dis = _dense_pre(h0, dp)
    h = h0
    for i in range(NLAYERS):
        theta = math.log(LAMDA / (i + 1) + 1.0)
        p = _spmm_sc(g, colp, rowp)
        outs = _dense_layer(theta, i == NLAYERS - 1, p, g, h0, dis, convs_w[i])
        g = outs[0]
        if i == NLAYERS - 1:
            h = outs[1]
    return _dense_final(h, fc1_w.T, fc1_b.reshape(1, NCLASS))


# speedup vs baseline: 4.1617x; 1.0703x over previous
import functools
import math

import jax
import jax.numpy as jnp
from jax import lax
from jax.experimental import pallas as pl
from jax.experimental.pallas import tpu as pltpu
from jax.experimental.pallas import tpu_sc as plsc

N = 10000
E = 320000
NFEAT = 128
NHID = 128
NCLASS = 64
NLAYERS = 8
LAMDA = 0.5
ALPHA = 0.1

NC = 2
NS = 16
FH = NHID // NC
CHUNK = 128
NBUF = 8
CPT = -(-E // (NS * CHUNK * NBUF)) * NBUF
EPT = CPT * CHUNK
E_PAD = EPT * NS
NCH = E_PAD // CHUNK
NGRP = CPT // NBUF
NACC = N + 8
RPT = 632
RPT_LAST_OUT = N - (NS - 1) * RPT
RPT_LAST_ZERO = NACC - (NS - 1) * RPT
LANES = 16


def _spmm_sc_body(
    gt_hbm, col_hbm, row_hbm, out_hbm, cidx_v, ridx_v, rows_v, zbuf_v, tab_sh, acc_sh, gsems, ssems
):
    c = lax.axis_index("c")
    s = lax.axis_index("s")

    zeros = jnp.zeros((2 * LANES,), jnp.bfloat16)

    def zero_row(i, _):
        for j in range(FH // (2 * LANES)):
            zbuf_v[i, pl.ds(j * 2 * LANES, 2 * LANES)] = zeros
        return 0

    lax.fori_loop(0, CHUNK, zero_row, 0)

    def prep_slab(base, nrows, nzero):
        nfull = nzero // CHUNK
        for k in range(nfull):
            pltpu.sync_copy(zbuf_v, acc_sh.at[pl.ds(base + k * CHUNK, CHUNK)])
        rem = nzero - nfull * CHUNK
        if rem:
            pltpu.sync_copy(
                zbuf_v.at[pl.ds(0, rem)], acc_sh.at[pl.ds(base + nfull * CHUNK, rem)]
            )
        pltpu.sync_copy(
            gt_hbm.at[pl.ds(base, nrows), pl.ds(c * FH, FH)], tab_sh.at[pl.ds(base, nrows)]
        )

    @pl.when(s < NS - 1)
    def _():
        prep_slab(s * RPT, RPT, RPT)

    @pl.when(s == NS - 1)
    def _():
        prep_slab((NS - 1) * RPT, RPT_LAST_OUT, RPT_LAST_ZERO)

    cbase = s * CPT
    pltpu.sync_copy(col_hbm.at[pl.ds(cbase, CPT)], cidx_v)
    pltpu.sync_copy(row_hbm.at[pl.ds(cbase, CPT)], ridx_v)

    plsc.subcore_barrier()

    def body(g, _):
        base = g * NBUF
        gd = [
            pltpu.async_copy(tab_sh.at[cidx_v.at[base + b]], rows_v.at[b], gsems[b])
            for b in range(NBUF)
        ]
        sd = []
        for b in range(NBUF):
            gd[b].wait()
            sd.append(
                pltpu.async_copy(
                    rows_v.at[b], acc_sh.at[ridx_v.at[base + b]], ssems[b], add=True
                )
            )
        for b in range(NBUF):
            sd[b].wait()
        return 0

    lax.fori_loop(0, NGRP, body, 0)

    plsc.subcore_barrier()

    @pl.when(s < NS - 1)
    def _():
        ob = s * RPT
        pltpu.sync_copy(
            acc_sh.at[pl.ds(ob, RPT)], out_hbm.at[pl.ds(ob, RPT), pl.ds(c * FH, FH)]
        )

    @pl.when(s == NS - 1)
    def _():
        ob = (NS - 1) * RPT
        pltpu.sync_copy(
            acc_sh.at[pl.ds(ob, RPT_LAST_OUT)],
            out_hbm.at[pl.ds(ob, RPT_LAST_OUT), pl.ds(c * FH, FH)],
        )


@functools.cache
def _get_spmm_sc():
    return pl.kernel(
        _spmm_sc_body,
        out_type=jax.ShapeDtypeStruct((N, NHID), jnp.bfloat16),
        mesh=plsc.VectorSubcoreMesh(
            core_axis_name="c", subcore_axis_name="s", num_cores=NC, num_subcores=NS
        ),
        scratch_types=[
            pltpu.VMEM((CPT, CHUNK), jnp.int32),
            pltpu.VMEM((CPT, CHUNK), jnp.int32),
            pltpu.VMEM((NBUF, CHUNK, FH), jnp.bfloat16),
            pltpu.VMEM((CHUNK, FH), jnp.bfloat16),
            pltpu.VMEM_SHARED((N, FH), jnp.bfloat16),
            pltpu.VMEM_SHARED((NACC, FH), jnp.bfloat16),
            [pltpu.SemaphoreType.DMA] * NBUF,
            [pltpu.SemaphoreType.DMA] * NBUF,
        ],
        compiler_params=pltpu.CompilerParams(use_tc_tiling_on_sc=False),
    )


def _spmm_sc(gt, colp, rowp):
    return _get_spmm_sc()(gt, colp, rowp)


DW = LANES
CPT_D = NCH // NC // NS


def _deg_sc_body(row_hbm, out_hbm, ridx_v, ones_v, zbuf_v, acc_sh):
    c = lax.axis_index("c")
    s = lax.axis_index("s")

    ones = jnp.full((LANES,), 1.0, jnp.float32)
    zeros = jnp.zeros((LANES,), jnp.float32)

    def fill_row(i, _):
        ones_v[i, pl.ds(0, LANES)] = ones
        zbuf_v[i, pl.ds(0, LANES)] = zeros
        return 0

    lax.fori_loop(0, CHUNK, fill_row, 0)

    def zero_slab(base, nzero):
        nfull = nzero // CHUNK
        for k in range(nfull):
            pltpu.sync_copy(zbuf_v, acc_sh.at[pl.ds(base + k * CHUNK, CHUNK)])
        rem = nzero - nfull * CHUNK
        if rem:
            pltpu.sync_copy(
                zbuf_v.at[pl.ds(0, rem)], acc_sh.at[pl.ds(base + nfull * CHUNK, rem)]
            )

    @pl.when(s < NS - 1)
    def _():
        zero_slab(s * RPT, RPT)

    @pl.when(s == NS - 1)
    def _():
        zero_slab((NS - 1) * RPT, RPT_LAST_ZERO)

    plsc.subcore_barrier()

    cbase = (c * NS + s) * CPT_D
    pltpu.sync_copy(row_hbm.at[pl.ds(cbase, CPT_D)], ridx_v)

    def body(k, _):
        pltpu.sync_copy(ones_v, acc_sh.at[ridx_v.at[k]], add=True)
        return 0

    lax.fori_loop(0, CPT_D, body, 0)

    plsc.subcore_barrier()

    @pl.when(s < NS - 1)
    def _():
        ob = s * RPT
        pltpu.sync_copy(acc_sh.at[pl.ds(ob, RPT)], out_hbm.at[c, pl.ds(ob, RPT)])

    @pl.when(s == NS - 1)
    def _():
        ob = (NS - 1) * RPT
        pltpu.sync_copy(
            acc_sh.at[pl.ds(ob, RPT_LAST_OUT)], out_hbm.at[c, pl.ds(ob, RPT_LAST_OUT)]
        )


@functools.cache
def _get_deg_sc():
    return pl.kernel(
        _deg_sc_body,
        out_type=jax.ShapeDtypeStruct((NC, N, DW), jnp.float32),
        mesh=plsc.VectorSubcoreMesh(
            core_axis_name="c", subcore_axis_name="s", num_cores=NC, num_subcores=NS
        ),
        scratch_types=[
            pltpu.VMEM((CPT_D, CHUNK), jnp.int32),
            pltpu.VMEM((CHUNK, DW), jnp.float32),
            pltpu.VMEM((CHUNK, DW), jnp.float32),
            pltpu.VMEM_SHARED((NACC, DW), jnp.float32),
        ],
        compiler_params=pltpu.CompilerParams(use_tc_tiling_on_sc=False),
    )


_BT = 2000


def _fc0_body(x_ref, w_ref, b_ref, h0_ref):
    h = jnp.dot(x_ref[...], w_ref[...], preferred_element_type=jnp.float32)
    h0_ref[...] = jnp.maximum(h + b_ref[...], 0.0)


def _dense_fc0(x, w_t, b):
    return pl.pallas_call(
        _fc0_body,
        grid=(N // _BT,),
        in_specs=[
            pl.BlockSpec((_BT, NFEAT), lambda i: (i, 0)),
            pl.BlockSpec((NFEAT, NHID), lambda i: (0, 0)),
            pl.BlockSpec((1, NHID), lambda i: (0, 0)),
        ],
        out_specs=pl.BlockSpec((_BT, NHID), lambda i: (i, 0)),
        out_shape=jax.ShapeDtypeStruct((N, NHID), jnp.float32),
    )(x, w_t, b)


def _pre_body(h0_ref, dp_ref, g_ref, dis_ref):
    deg = dp_ref[0, :, 0:1] + dp_ref[1, :, 0:1] + 1.0
    dis = lax.rsqrt(deg)
    dis_ref[...] = dis
    g_ref[...] = (h0_ref[...] * dis).astype(jnp.bfloat16)


def _dense_pre(h0, dp):
    return pl.pallas_call(
        _pre_body,
        grid=(N // _BT,),
        in_specs=[
            pl.BlockSpec((_BT, NHID), lambda i: (i, 0)),
            pl.BlockSpec((NC, _BT, DW), lambda i: (0, i, 0)),
        ],
        out_specs=[
            pl.BlockSpec((_BT, NHID), lambda i: (i, 0)),
            pl.BlockSpec((_BT, 1), lambda i: (i, 0)),
        ],
        out_shape=[
            jax.ShapeDtypeStruct((N, NHID), jnp.bfloat16),
            jax.ShapeDtypeStruct((N, 1), jnp.float32),
        ],
    )(h0, dp)


def _layer_body(theta, emit_h, p_ref, g_ref, h0_ref, dis_ref, w_ref, *out_refs):
    dis = dis_ref[...]
    msgs = p_ref[...].astype(jnp.float32) + g_ref[...].astype(jnp.float32)
    s = (1.0 - ALPHA) * (dis * msgs) + ALPHA * h0_ref[...]
    mm = jnp.dot(s, w_ref[...], preferred_element_type=jnp.float32)
    h = jnp.maximum(theta * mm + (1.0 - theta) * s, 0.0)
    out_refs[0][...] = (h * dis).astype(jnp.bfloat16)
    if emit_h:
        out_refs[1][...] = h


def _dense_layer(theta, emit_h, p, g, h0, dis, w):
    out_specs = [pl.BlockSpec((_BT, NHID), lambda i: (i, 0))]
    out_shape = [jax.ShapeDtypeStruct((N, NHID), jnp.bfloat16)]
    if emit_h:
        out_specs.append(pl.BlockSpec((_BT, NHID), lambda i: (i, 0)))
        out_shape.append(jax.ShapeDtypeStruct((N, NHID), jnp.float32))
    return pl.pallas_call(
        functools.partial(_layer_body, theta, emit_h),
        grid=(N // _BT,),
        in_specs=[
            pl.BlockSpec((_BT, NHID), lambda i: (i, 0)),
            pl.BlockSpec((_BT, NHID), lambda i: (i, 0)),
            pl.BlockSpec((_BT, NHID), lambda i: (i, 0)),
            pl.BlockSpec((_BT, 1), lambda i: (i, 0)),
            pl.BlockSpec((NHID, NHID), lambda i: (0, 0)),
        ],
        out_specs=out_specs,
        out_shape=out_shape,
    )(p, g, h0, dis, w)


def _final_body(h_ref, w_ref, b_ref, o_ref):
    o = jnp.dot(h_ref[...], w_ref[...], preferred_element_type=jnp.float32)
    o_ref[...] = o + b_ref[...]


def _dense_final(h, w_t, b):
    return pl.pallas_call(
        _final_body,
        grid=(N // _BT,),
        in_specs=[
            pl.BlockSpec((_BT, NHID), lambda i: (i, 0)),
            pl.BlockSpec((NHID, NCLASS), lambda i: (0, 0)),
            pl.BlockSpec((1, NCLASS), lambda i: (0, 0)),
        ],
        out_specs=pl.BlockSpec((_BT, NCLASS), lambda i: (i, 0)),
        out_shape=jax.ShapeDtypeStruct((N, NCLASS), jnp.float32),
    )(h, w_t, b)


def kernel(x, edge_idx, fc0_w, fc0_b, convs_w, fc1_w, fc1_b):
    row = edge_idx[0].astype(jnp.int32)
    col = edge_idx[1].astype(jnp.int32)
    self_m = row == col

    row_r = jnp.where(self_m, N, row)
    rowf = jnp.concatenate([row_r, jnp.full((E_PAD - E,), N, jnp.int32)])
    colf = jnp.concatenate([col, jnp.zeros((E_PAD - E,), jnp.int32)])
    rowp = rowf.reshape(NCH, CHUNK)
    colp = colf.reshape(NCH, CHUNK)

    dp = _get_deg_sc()(rowp)
    h0 = _dense_fc0(x, fc0_w.T, fc0_b.reshape(1, NHID))
    g, dis = _dense_pre(h0, dp)
    h = h0
    for i in range(NLAYERS):
        theta = math.log(LAMDA / (i + 1) + 1.0)
        p = _spmm_sc(g, colp, rowp)
        outs = _dense_layer(theta, i == NLAYERS - 1, p, g, h0, dis, convs_w[i])
        g = outs[0]
        if i == NLAYERS - 1:
            h = outs[1]
    return _dense_final(h, fc1_w.T, fc1_b.reshape(1, NCLASS))
